# Initial kernel scaffold; baseline (speedup 1.0000x reference)
import jax
import jax.numpy as jnp
from jax.experimental import pallas as pl


def kernel(x, last_update, edge_index, t, msg, Wt, bt, Wl1, Wr1, We1, att1, b1, Wl2, Wr2, We2, att2, b2):
    raise NotImplementedError("write your pallas kernel here")


# speedup vs baseline: 10.4143x; 10.4143x over previous
import functools

import jax
import jax.numpy as jnp
from jax import lax
from jax.experimental import pallas as pl
from jax.experimental.pallas import tpu as pltpu
from jax.experimental.pallas import tpu_sc as plsc

N = 10000
NPAD = 10240
E = 320000
EP = E + N
EPAD1 = 331776
EPADB = 323584
C = 128
CA = 80
CB = C - CA
TENC = 32
EDIM = 48
BLK = 128
NT = 16
NCORE = 2
L = 16

_SC_PARAMS = pltpu.CompilerParams(use_tc_tiling_on_sc=False,
                                  needs_layout_passes=False)


def _mesh():
    return plsc.VectorSubcoreMesh(core_axis_name="c", subcore_axis_name="s")


def _rel_t_kernel():
    nblk = EPADB // (NCORE * NT) // BLK

    @functools.partial(
        pl.kernel,
        out_type=jax.ShapeDtypeStruct((EPADB,), jnp.float32),
        mesh=_mesh(),
        scratch_types=[
            pltpu.VMEM((N,), jnp.float32),
            pltpu.VMEM((BLK,), jnp.int32),
            pltpu.VMEM((BLK,), jnp.float32),
            pltpu.VMEM((BLK,), jnp.float32),
        ],
        compiler_params=_SC_PARAMS,
    )
    def k(lu, srcp, tp, out, lu_v, idx_v, t_v, rel_v):
        c = lax.axis_index("c")
        s = lax.axis_index("s")
        base = (c * NT + s) * (nblk * BLK)
        pltpu.sync_copy(lu, lu_v)

        def blk(b, carry):
            off = base + b * BLK
            pltpu.sync_copy(srcp.at[pl.ds(off, BLK)], idx_v)
            pltpu.sync_copy(tp.at[pl.ds(off, BLK)], t_v)
            for g in range(BLK // L):
                vals = plsc.load_gather(lu_v, [idx_v[pl.ds(g * L, L)]])
                rel_v[pl.ds(g * L, L)] = vals - t_v[pl.ds(g * L, L)]
            pltpu.sync_copy(rel_v, out.at[pl.ds(off, BLK)])
            return carry

        lax.fori_loop(0, nblk, blk, None)

    return k


def _attr_scatter_kernel():
    nblk = EPADB // (NCORE * NT) // BLK
    npt = NPAD // NT

    @functools.partial(
        pl.kernel,
        out_type=(
            jax.ShapeDtypeStruct((NCORE, NPAD, EDIM), jnp.float32),
            jax.ShapeDtypeStruct((NCORE, NT, NPAD // L, L), jnp.float32),
        ),
        mesh=_mesh(),
        scratch_types=[
            pltpu.VMEM_SHARED((NPAD, EDIM), jnp.float32),
            pltpu.VMEM((NPAD // L, L), jnp.float32),
            pltpu.VMEM((BLK,), jnp.int32),
            pltpu.VMEM((BLK, EDIM), jnp.float32),
        ],
        compiler_params=_SC_PARAMS,
    )
    def k(dstp, attr, attr_sum, deg, acc, deg_t, dstv, rows):
        c = lax.axis_index("c")
        s = lax.axis_index("s")
        base = (c * NT + s) * (nblk * BLK)
        z16 = jnp.zeros((L,), jnp.float32)
        ones16 = jnp.ones((L,), jnp.float32)

        def zrow(r, carry):
            for kk in range(EDIM // L):
                rows[r, pl.ds(kk * L, L)] = z16
            return carry

        lax.fori_loop(0, BLK, zrow, None)

        def zdeg(i, carry):
            deg_t[i, :] = z16
            return carry

        lax.fori_loop(0, NPAD // L, zdeg, None)

        def zacc(i, carry):
            pltpu.sync_copy(rows, acc.at[pl.ds(s * npt + i * BLK, BLK)])
            return carry

        lax.fori_loop(0, npt // BLK, zacc, None)
        plsc.subcore_barrier()

        def blk(b, carry):
            off = base + b * BLK
            pltpu.sync_copy(dstp.at[pl.ds(off, BLK)], dstv)
            pltpu.sync_copy(attr.at[pl.ds(off, BLK)], rows)
            pltpu.sync_copy(rows, acc.at[dstv], add=True)
            for g in range(BLK // L):
                dv = dstv[pl.ds(g * L, L)]
                plsc.addupdate_scatter(deg_t, [dv >> 4, dv & 15], ones16)
            return carry

        lax.fori_loop(0, nblk, blk, None)
        plsc.subcore_barrier()

        def wout(i, carry):
            r0 = s * npt + i * BLK
            pltpu.sync_copy(acc.at[pl.ds(r0, BLK)], attr_sum.at[c, pl.ds(r0, BLK)])
            return carry

        lax.fori_loop(0, npt // BLK, wout, None)
        pltpu.sync_copy(deg_t, deg.at[c, s])

    return k


def _edge_logits_kernel(H):
    if H == 2:
        nblk = EPAD1 // NT // BLK
    else:
        nblk = EPAD1 // (NCORE * NT) // BLK
    npt = NPAD // NT
    NK = C // L
    NA = CA // L

    @functools.partial(
        pl.kernel,
        out_type=(
            jax.ShapeDtypeStruct((NCORE, NPAD, CA), jnp.float32),
            jax.ShapeDtypeStruct((NCORE, NT, NPAD // L, L), jnp.float32),
            jax.ShapeDtypeStruct((NCORE if H == 2 else 1, EPAD1), jnp.float32),
        ),
        mesh=_mesh(),
        scratch_types=[
            pltpu.VMEM_SHARED((NPAD, CA), jnp.float32),
            pltpu.VMEM((NPAD // L, L), jnp.float32),
            pltpu.VMEM((BLK,), jnp.int32),
            pltpu.VMEM((BLK,), jnp.int32),
            pltpu.VMEM((BLK,), jnp.int32),
            pltpu.VMEM((BLK,), jnp.int32),
            pltpu.VMEM((BLK, C), jnp.float32),
            pltpu.VMEM((BLK, C), jnp.float32),
            pltpu.VMEM((BLK, C), jnp.float32),
            pltpu.VMEM((BLK, CA), jnp.float32),
            pltpu.VMEM((BLK,), jnp.float32),
            pltpu.VMEM((H, C), jnp.float32),
            pltpu.SemaphoreType.DMA,
            pltpu.SemaphoreType.DMA,
        ],
        compiler_params=_SC_PARAMS,
    )
    def k(srcp, dstp, xl, xr, ef, att, num, den, w_out,
          acc, den_t, srcv, dstv, gidx, didx, gl, gr, eb, obuf, wbuf, attb,
          sem1, sem2):
        c = lax.axis_index("c")
        s = lax.axis_index("s")
        if H == 2:
            base = s * (nblk * BLK)
            e_off = c * EPAD1
        else:
            base = (c * NT + s) * (nblk * BLK)
            e_off = 0
        z16 = jnp.zeros((L,), jnp.float32)

        pltpu.sync_copy(att, attb)

        def zrow(r, carry):
            for kk in range(NA):
                obuf[r, pl.ds(kk * L, L)] = z16
            return carry

        lax.fori_loop(0, BLK, zrow, None)

        def zdeg(i, carry):
            den_t[i, :] = z16
            return carry

        lax.fori_loop(0, NPAD // L, zdeg, None)

        def zacc(i, carry):
            pltpu.sync_copy(obuf, acc.at[pl.ds(s * npt + i * BLK, BLK)])
            return carry

        lax.fori_loop(0, npt // BLK, zacc, None)
        plsc.subcore_barrier()

        def blk_body(b, carry):
            off = base + b * BLK
            pltpu.sync_copy(srcp.at[pl.ds(off, BLK)], srcv)
            pltpu.sync_copy(dstp.at[pl.ds(off, BLK)], dstv)
            if H == 2:
                offv = jnp.broadcast_to(c * NPAD, (L,)).astype(jnp.int32)
                for g in range(BLK // L):
                    gidx[pl.ds(g * L, L)] = srcv[pl.ds(g * L, L)] + offv
                    didx[pl.ds(g * L, L)] = dstv[pl.ds(g * L, L)] + offv
                gsrc, gdst = gidx, didx
            else:
                gsrc, gdst = srcv, dstv
            cp1 = pltpu.async_copy(xl.at[gsrc], gl, sem1)
            cp2 = pltpu.async_copy(xr.at[gdst], gr, sem2)
            pltpu.sync_copy(ef.at[pl.ds(e_off + off, BLK)], eb)
            cp1.wait()
            cp2.wait()

            hrow = c if H == 2 else 0
            attk = [attb[hrow, pl.ds(kk * L, L)] for kk in range(NK)]
            iot = lax.iota(jnp.int32, L)

            def grp(g, carry):
                r0 = g * L
                wacc = z16
                for j in range(L):
                    r = r0 + j
                    glk = [gl[r, pl.ds(kk * L, L)] for kk in range(NK)]
                    accv = None
                    for kk in range(NK):
                        u = glk[kk] + gr[r, pl.ds(kk * L, L)] + eb[r, pl.ds(kk * L, L)]
                        lr = jnp.where(u >= 0.0, u, 0.2 * u)
                        term = lr * attk[kk]
                        accv = term if accv is None else accv + term
                    tot = jnp.sum(accv)
                    wv = jnp.exp(jnp.broadcast_to(tot, (L,)))
                    for kk in range(NA):
                        obuf[r, pl.ds(kk * L, L)] = wv * glk[kk]
                    wacc = jnp.where(iot == j, wv, wacc)
                wbuf[pl.ds(r0, L)] = wacc
                dv = dstv[pl.ds(r0, L)]
                plsc.addupdate_scatter(den_t, [dv >> 4, dv & 15], wacc)
                return carry

            lax.fori_loop(0, BLK // L, grp, None)
            pltpu.sync_copy(obuf, acc.at[dstv], add=True)
            if H == 2:
                pltpu.sync_copy(wbuf, w_out.at[c, pl.ds(off, BLK)])
            else:
                pltpu.sync_copy(wbuf, w_out.at[0, pl.ds(off, BLK)])
            return carry

        lax.fori_loop(0, nblk, blk_body, None)
        plsc.subcore_barrier()

        def wout(i, carry):
            r0 = s * npt + i * BLK
            pltpu.sync_copy(acc.at[pl.ds(r0, BLK)], num.at[c, pl.ds(r0, BLK)])
            return carry

        lax.fori_loop(0, npt // BLK, wout, None)
        pltpu.sync_copy(den_t, den.at[c, s])

    return k


def _edge_apply_kernel(H):
    if H == 2:
        nblk = EPAD1 // NT // BLK
    else:
        nblk = EPAD1 // (NCORE * NT) // BLK
    npt = NPAD // NT
    NB = CB // L

    @functools.partial(
        pl.kernel,
        out_type=jax.ShapeDtypeStruct((NCORE, NPAD, CB), jnp.float32),
        mesh=_mesh(),
        scratch_types=[
            pltpu.VMEM_SHARED((NPAD, CB), jnp.float32),
            pltpu.VMEM((BLK,), jnp.int32),
            pltpu.VMEM((BLK,), jnp.int32),
            pltpu.VMEM((BLK,), jnp.int32),
            pltpu.VMEM((BLK, CB), jnp.float32),
            pltpu.VMEM((BLK,), jnp.float32),
            pltpu.SemaphoreType.DMA,
        ],
        compiler_params=_SC_PARAMS,
    )
    def k(srcp, dstp, xlB, w_in, num, acc, srcv, dstv, gidx, glB, wbuf, sem):
        c = lax.axis_index("c")
        s = lax.axis_index("s")
        if H == 2:
            base = s * (nblk * BLK)
        else:
            base = (c * NT + s) * (nblk * BLK)
        z16 = jnp.zeros((L,), jnp.float32)

        def zrow(r, carry):
            for kk in range(NB):
                glB[r, pl.ds(kk * L, L)] = z16
            return carry

        lax.fori_loop(0, BLK, zrow, None)

        def zacc(i, carry):
            pltpu.sync_copy(glB, acc.at[pl.ds(s * npt + i * BLK, BLK)])
            return carry

        lax.fori_loop(0, npt // BLK, zacc, None)
        plsc.subcore_barrier()

        iot = lax.iota(jnp.int32, L)

        def blk_body(b, carry):
            off = base + b * BLK
            pltpu.sync_copy(srcp.at[pl.ds(off, BLK)], srcv)
            pltpu.sync_copy(dstp.at[pl.ds(off, BLK)], dstv)
            if H == 2:
                pltpu.sync_copy(w_in.at[c, pl.ds(off, BLK)], wbuf)
                offv = jnp.broadcast_to(c * NPAD, (L,)).astype(jnp.int32)
                for g in range(BLK // L):
                    gidx[pl.ds(g * L, L)] = srcv[pl.ds(g * L, L)] + offv
                gsrc = gidx
            else:
                pltpu.sync_copy(w_in.at[0, pl.ds(off, BLK)], wbuf)
                gsrc = srcv
            pltpu.async_copy(xlB.at[gsrc], glB, sem).wait()

            def grp(g, carry):
                r0 = g * L
                w16 = wbuf[pl.ds(r0, L)]
                for j in range(L):
                    r = r0 + j
                    tot = jnp.sum(jnp.where(iot == j, w16, z16))
                    wv = jnp.broadcast_to(tot, (L,))
                    for kk in range(NB):
                        glB[r, pl.ds(kk * L, L)] = wv * glB[r, pl.ds(kk * L, L)]
                return carry

            lax.fori_loop(0, BLK // L, grp, None)
            pltpu.sync_copy(glB, acc.at[dstv], add=True)
            return carry

        lax.fori_loop(0, nblk, blk_body, None)
        plsc.subcore_barrier()

        def wout(i, carry):
            r0 = s * npt + i * BLK
            pltpu.sync_copy(acc.at[pl.ds(r0, BLK)], num.at[c, pl.ds(r0, BLK)])
            return carry

        lax.fori_loop(0, npt // BLK, wout, None)

    return k


_REL_K = _rel_t_kernel()
_ATTR_K = _attr_scatter_kernel()
_LOGITS_K2 = _edge_logits_kernel(2)
_LOGITS_K1 = _edge_logits_kernel(1)
_APPLY_K2 = _edge_apply_kernel(2)
_APPLY_K1 = _edge_apply_kernel(1)


def _mm_headed(A, W, H, blk_rows):
    M, K = A.shape

    def kern(a_ref, w_ref, o_ref):
        o_ref[0] = jnp.dot(a_ref[...], w_ref[...],
                           preferred_element_type=jnp.float32)

    return pl.pallas_call(
        kern,
        grid=(H, M // blk_rows),
        in_specs=[
            pl.BlockSpec((blk_rows, K), lambda h, i: (i, 0)),
            pl.BlockSpec((K, 128), lambda h, i: (0, h)),
        ],
        out_specs=pl.BlockSpec((1, blk_rows, 128), lambda h, i: (h, i, 0)),
        out_shape=jax.ShapeDtypeStruct((H, M, 128), jnp.float32),
    )(A, W)


def _edge_attr(rel_col, msg, Wt, bt):
    blk = 3200

    def kern(r_ref, m_ref, wt_ref, bt_ref, o_ref):
        enc = jnp.cos(r_ref[...] * wt_ref[...] + bt_ref[...])
        o_ref[...] = jnp.concatenate([enc, m_ref[...]], axis=1)

    return pl.pallas_call(
        kern,
        grid=(E // blk,),
        in_specs=[
            pl.BlockSpec((blk, 1), lambda i: (i, 0)),
            pl.BlockSpec((blk, 16), lambda i: (i, 0)),
            pl.BlockSpec((1, TENC), lambda i: (0, 0)),
            pl.BlockSpec((1, TENC), lambda i: (0, 0)),
        ],
        out_specs=pl.BlockSpec((blk, EDIM), lambda i: (i, 0)),
        out_shape=jax.ShapeDtypeStruct((E, EDIM), jnp.float32),
    )(rel_col, msg, Wt, bt)


def _attr_mean(attr_sum, deg):
    blk = 1280

    def kern(a_ref, d_ref, o_ref):
        asum = a_ref[0] + a_ref[1]
        dsum = jnp.sum(d_ref[...], axis=(0, 1))
        o_ref[...] = asum / jnp.clip(dsum, 1.0, None)[:, None]

    return pl.pallas_call(
        kern,
        grid=(NPAD // blk,),
        in_specs=[
            pl.BlockSpec((2, blk, EDIM), lambda i: (0, i, 0)),
            pl.BlockSpec((2, NT, blk), lambda i: (0, 0, i)),
        ],
        out_specs=pl.BlockSpec((blk, EDIM), lambda i: (i, 0)),
        out_shape=jax.ShapeDtypeStruct((NPAD, EDIM), jnp.float32),
    )(attr_sum, deg)


def _div1(numA, numB, den, b1):
    blk = 1280

    def kern(na_ref, nb_ref, d_ref, b_ref, o_ref):
        i = pl.program_id(0)
        de = jnp.sum(d_ref[...], axis=1)
        de = jnp.where(de == 0.0, 1.0, de)
        h0 = jnp.concatenate([na_ref[0], nb_ref[0]], axis=1) / de[0][:, None]
        h1 = jnp.concatenate([na_ref[1], nb_ref[1]], axis=1) / de[1][:, None]
        h = jnp.concatenate([h0, h1], axis=1) + b_ref[...]
        h = jnp.maximum(h, 0.0)
        grow = i * blk + lax.broadcasted_iota(jnp.int32, (blk, 1), 0)
        o_ref[...] = jnp.where(grow < N, h, 0.0)

    return pl.pallas_call(
        kern,
        grid=(NPAD // blk,),
        in_specs=[
            pl.BlockSpec((2, blk, CA), lambda i: (0, i, 0)),
            pl.BlockSpec((2, blk, CB), lambda i: (0, i, 0)),
            pl.BlockSpec((2, NT, blk), lambda i: (0, 0, i)),
            pl.BlockSpec((1, 2 * C), lambda i: (0, 0)),
        ],
        out_specs=pl.BlockSpec((blk, 2 * C), lambda i: (i, 0)),
        out_shape=jax.ShapeDtypeStruct((NPAD, 2 * C), jnp.float32),
    )(numA, numB, den, b1)


def _div2(numA, numB, den, b2):
    blk = 1280

    def kern(na_ref, nb_ref, d_ref, b_ref, o_ref):
        de = jnp.sum(d_ref[...], axis=(0, 1))
        de = jnp.where(de == 0.0, 1.0, de)
        nsum = jnp.concatenate([na_ref[0] + na_ref[1], nb_ref[0] + nb_ref[1]],
                               axis=1)
        o_ref[...] = nsum / de[:, None] + b_ref[...]

    return pl.pallas_call(
        kern,
        grid=(NPAD // blk,),
        in_specs=[
            pl.BlockSpec((2, blk, CA), lambda i: (0, i, 0)),
            pl.BlockSpec((2, blk, CB), lambda i: (0, i, 0)),
            pl.BlockSpec((2, NT, blk), lambda i: (0, 0, i)),
            pl.BlockSpec((1, C), lambda i: (0, 0)),
        ],
        out_specs=pl.BlockSpec((blk, C), lambda i: (i, 0)),
        out_shape=jax.ShapeDtypeStruct((NPAD, C), jnp.float32),
    )(numA, numB, den, b2)


def kernel(x, last_update, edge_index, t, msg, Wt, bt,
           Wl1, Wr1, We1, att1, b1, Wl2, Wr2, We2, att2, b2):
    f32 = jnp.float32
    i32 = jnp.int32
    src = edge_index[0].astype(i32)
    dst = edge_index[1].astype(i32)

    srcA = jnp.concatenate([src, jnp.zeros((EPADB - E,), i32)])
    tA = jnp.concatenate([t, jnp.zeros((EPADB - E,), f32)])
    rel = _REL_K(last_update, srcA, tA)
    rel_col = rel[:E, None]

    ea = _edge_attr(rel_col, msg, Wt, bt.reshape(1, TENC))

    dstB = jnp.concatenate([dst, jnp.full((EPADB - E,), N, i32)])
    eaB = jnp.concatenate([ea, jnp.zeros((EPADB - E, EDIM), f32)])
    attr_sum, degB = _ATTR_K(dstB, eaB)
    attr_mean = _attr_mean(attr_sum, degB.reshape(NCORE, NT, NPAD))

    loop_idx = jnp.arange(N, dtype=i32)
    padE = EPAD1 - EP
    src2 = jnp.concatenate([src, loop_idx, jnp.zeros((padE,), i32)])
    dst2 = jnp.concatenate([dst, loop_idx, jnp.full((padE,), N, i32)])
    ea_full = jnp.concatenate([ea, attr_mean[:N], jnp.zeros((padE, EDIM), f32)])

    x_pad = jnp.pad(x, ((0, NPAD - N), (0, 0)))
    xl1 = _mm_headed(x_pad, Wl1, 2, 1280).reshape(2 * NPAD, C)
    xr1 = _mm_headed(x_pad, Wr1, 2, 1280).reshape(2 * NPAD, C)
    e1 = _mm_headed(ea_full, We1, 2, 4096).reshape(2 * EPAD1, C)
    xl1B = xl1[:, CA:]

    numA1, den1, w1 = _LOGITS_K2(src2, dst2, xl1, xr1, e1, att1)
    numB1 = _APPLY_K2(src2, dst2, xl1B, w1)
    h = _div1(numA1, numB1, den1.reshape(NCORE, NT, NPAD), b1.reshape(1, 2 * C))

    xl2 = _mm_headed(h, Wl2, 1, 1280).reshape(NPAD, C)
    xr2 = _mm_headed(h, Wr2, 1, 1280).reshape(NPAD, C)
    e2 = _mm_headed(ea_full, We2, 1, 4096).reshape(EPAD1, C)
    xl2B = xl2[:, CA:]

    numA2, den2, w2 = _LOGITS_K1(src2, dst2, xl2, xr2, e2, att2)
    numB2 = _APPLY_K1(src2, dst2, xl2B, w2)
    out = _div2(numA2, numB2, den2.reshape(NCORE, NT, NPAD), b2.reshape(1, C))
    return out[:N]
